# baseline (speedup 1.0000x reference)
import functools

import jax
import jax.numpy as jnp
from jax import lax
from jax.experimental import pallas as pl
from jax.experimental.pallas import tpu as pltpu
from jax.experimental.pallas import tpu_sc as plsc

E = 8
TOP_K = 2
D_IN = 1024
D_HID = 512
D_OUT = 1024
N = 2048
TILE = 2048
NT = N // TILE
S = N * TOP_K
SR, SL = 32, 128
PT = 512
NPT = S // PT

NW = 32
ROWS_PER_W = S // NW
CHUNK = 32
NCH = ROWS_PER_W // CHUNK

_NEG_INF = float("-inf")


def _gate_body(feat_ref, gw1_ref, gw2_ref, gb1_ref, gb2_ref, rec_ref,
               tki_ref, tkw_ref, ranks_ref, totals_ref, gh_acc):
    i = pl.program_id(0)
    e = pl.program_id(1)
    part = jnp.dot(feat_ref[0], gw1_ref[e], preferred_element_type=jnp.float32)

    @pl.when(e == 0)
    def _():
        gh_acc[...] = part

    @pl.when(e > 0)
    def _():
        gh_acc[...] = gh_acc[...] + part

    @pl.when(e == E - 1)
    def _():
        gh = jax.nn.relu(gh_acc[...] + gb1_ref[...])
        logits = jnp.dot(gh, gw2_ref[...],
                         preferred_element_type=jnp.float32) + gb2_ref[...]
        m = jnp.max(logits, axis=1, keepdims=True)
        ex = jnp.exp(logits - m)
        sm = ex / jnp.sum(ex, axis=1, keepdims=True)
        scores = sm + rec_ref[...]

        iota = lax.broadcasted_iota(jnp.int32, (TILE, E), 1)
        v1 = jnp.max(scores, axis=1, keepdims=True)
        i1 = jnp.min(jnp.where(scores == v1, iota, E), axis=1, keepdims=True)
        masked = jnp.where(iota == i1, _NEG_INF, scores)
        v2 = jnp.max(masked, axis=1, keepdims=True)
        i2 = jnp.min(jnp.where(masked == v2, iota, E), axis=1, keepdims=True)
        s = v1 + v2
        w_1 = v1 / s
        w_2 = v2 / s

        tki_ref[...] = jnp.concatenate([i1, i2], axis=1)
        tkw_ref[...] = jnp.concatenate([w_1, w_2], axis=1)
        oh1 = (iota == i1)
        oh2 = (iota == i2)
        r = 2 - 2 * oh1.astype(jnp.int32) - oh2.astype(jnp.int32)
        ranks_ref[...] = r
        part_tot = jnp.sum(r, axis=0, keepdims=True)

        @pl.when(i == 0)
        def _():
            totals_ref[...] = part_tot

        @pl.when(i > 0)
        def _():
            totals_ref[...] = totals_ref[...] + part_tot


def _route_body(eid_ref, posm_ref, srcm_ref, offs_ref):
    eid = eid_ref[...]
    jl = lax.broadcasted_iota(jnp.int32, (SL, SL), 1)
    kl = lax.broadcasted_iota(jnp.int32, (SL, SL), 0)
    t128 = (kl <= jl).astype(jnp.float32)
    rr = lax.broadcasted_iota(jnp.int32, (SR, SR), 0)
    rc = lax.broadcasted_iota(jnp.int32, (SR, SR), 1)
    t32s = (rc < rr).astype(jnp.float32)

    p1 = []
    scols = []
    for e in range(E):
        oh = (eid == e).astype(jnp.float32)
        pe = jnp.dot(oh, t128, preferred_element_type=jnp.float32)
        p1.append(pe)
        scols.append(pe[:, SL - 1:SL])
    smat = jnp.concatenate(scols, axis=1)
    omat = jnp.dot(t32s, smat, preferred_element_type=jnp.float32)
    counts = jnp.sum(smat, axis=0, keepdims=True)

    offs_list = [jnp.zeros((1, 1), jnp.float32)]
    for e in range(1, E):
        offs_list.append(offs_list[e - 1] + counts[:, e - 1:e])
    offs = jnp.concatenate(offs_list, axis=1)

    pos = jnp.zeros((SR, SL), jnp.float32)
    for e in range(E):
        oh = (eid == e).astype(jnp.float32)
        pos = pos + oh * (p1[e] - 1.0 + omat[:, e:e + 1] + offs[:, e:e + 1])
    posm_ref[...] = pos.astype(jnp.int32)

    ri = lax.broadcasted_iota(jnp.int32, (SR, SL), 0)
    ji = lax.broadcasted_iota(jnp.int32, (SR, SL), 1)
    tok = (ri * SL + ji) // TOP_K
    srcm_ref[...] = eid * N + tok
    offs_ref[...] = offs.astype(jnp.int32)


def _sc_dispatch(table, gidx3, sidx3):
    mesh = plsc.VectorSubcoreMesh(core_axis_name="c", subcore_axis_name="s")

    @functools.partial(
        pl.kernel, mesh=mesh,
        out_type=jax.ShapeDtypeStruct((S, D_IN), jnp.float32),
        scratch_types=[
            pltpu.VMEM((NCH, CHUNK), jnp.int32),
            pltpu.VMEM((NCH, CHUNK), jnp.int32),
            pltpu.VMEM((CHUNK, D_IN), jnp.float32),
            pltpu.VMEM((CHUNK, D_IN), jnp.float32),
            pltpu.VMEM((CHUNK, D_IN), jnp.float32),
            pltpu.SemaphoreType.DMA,
            pltpu.SemaphoreType.DMA,
            pltpu.SemaphoreType.DMA,
            pltpu.SemaphoreType.DMA,
            pltpu.SemaphoreType.DMA,
            pltpu.SemaphoreType.DMA,
        ],
    )
    def k(table_hbm, gidx_hbm, sidx_hbm, out_hbm,
          gidx_v, sidx_v, b0, b1, b2, g0, g1, g2, s0, s1, s2):
        wid = lax.axis_index("s") * 2 + lax.axis_index("c")
        pltpu.sync_copy(gidx_hbm.at[wid], gidx_v)
        pltpu.sync_copy(sidx_hbm.at[wid], sidx_v)
        bufs = (b0, b1, b2)
        gsems = (g0, g1, g2)
        ssems = (s0, s1, s2)
        gcps = [None] * 3
        scps = [None] * 3
        for c in range(NCH):
            b = c % 3
            if c >= 3:
                scps[b].wait()
            gcps[b] = pltpu.async_copy(table_hbm.at[gidx_v.at[c]], bufs[b],
                                       gsems[b])
            if c >= 1:
                pb = (c - 1) % 3
                gcps[pb].wait()
                scps[pb] = pltpu.async_copy(bufs[pb],
                                            out_hbm.at[sidx_v.at[c - 1]],
                                            ssems[pb])
        lb = (NCH - 1) % 3
        gcps[lb].wait()
        scps[lb] = pltpu.async_copy(bufs[lb], out_hbm.at[sidx_v.at[NCH - 1]],
                                    ssems[lb])
        for c in range(max(0, NCH - 3), NCH):
            scps[c % 3].wait()

    return k(table, gidx3, sidx3)


def _sc_gather_rows(table, idx3, out_rows, out_cols):
    mesh = plsc.VectorSubcoreMesh(core_axis_name="c", subcore_axis_name="s")

    @functools.partial(
        pl.kernel, mesh=mesh,
        out_type=jax.ShapeDtypeStruct((out_rows, out_cols), jnp.float32),
        scratch_types=[
            pltpu.VMEM((NCH, CHUNK), jnp.int32),
            pltpu.VMEM((CHUNK, out_cols), jnp.float32),
            pltpu.VMEM((CHUNK, out_cols), jnp.float32),
            pltpu.VMEM((CHUNK, out_cols), jnp.float32),
            pltpu.SemaphoreType.DMA,
            pltpu.SemaphoreType.DMA,
            pltpu.SemaphoreType.DMA,
            pltpu.SemaphoreType.DMA,
            pltpu.SemaphoreType.DMA,
            pltpu.SemaphoreType.DMA,
        ],
    )
    def k(table_hbm, idx_hbm, out_hbm, idx_v, b0, b1, b2,
          g0, g1, g2, w0, w1, w2):
        wid = lax.axis_index("s") * 2 + lax.axis_index("c")
        base = wid * ROWS_PER_W
        pltpu.sync_copy(idx_hbm.at[wid], idx_v)
        bufs = (b0, b1, b2)
        gsems = (g0, g1, g2)
        wsems = (w0, w1, w2)
        gcps = [None] * 3
        wcps = [None] * 3
        for c in range(NCH):
            b = c % 3
            if c >= 3:
                wcps[b].wait()
            gcps[b] = pltpu.async_copy(table_hbm.at[idx_v.at[c]], bufs[b],
                                       gsems[b])
            if c >= 1:
                pb = (c - 1) % 3
                gcps[pb].wait()
                wcps[pb] = pltpu.async_copy(
                    bufs[pb], out_hbm.at[pl.ds(base + (c - 1) * CHUNK, CHUNK)],
                    wsems[pb])
        lb = (NCH - 1) % 3
        gcps[lb].wait()
        wcps[lb] = pltpu.async_copy(
            bufs[lb], out_hbm.at[pl.ds(base + (NCH - 1) * CHUNK, CHUNK)],
            wsems[lb])
        for c in range(max(0, NCH - 3), NCH):
            wcps[c % 3].wait()

    return k(table, idx3)


def _grouped_body(offs_ref, cin_ref, w1_ref, b1_ref, w2_ref, b2_ref, out_ref,
                  acc):
    t = pl.program_id(0)
    pid = t * PT + lax.broadcasted_iota(jnp.int32, (PT, 1), 0)
    x = cin_ref[...]
    acc[...] = jnp.zeros((PT, D_OUT), jnp.float32)
    for e in range(E):
        start = offs_ref[e]
        end = offs_ref[e + 1]
        active = jnp.logical_and(start < (t + 1) * PT, end > t * PT)

        @pl.when(active)
        def _(e=e, start=start, end=end):
            h = jax.nn.relu(
                jnp.dot(x, w1_ref[e],
                        preferred_element_type=jnp.float32) + b1_ref[e])
            o = jnp.dot(h, w2_ref[e],
                        preferred_element_type=jnp.float32) + b2_ref[e]
            mask = jnp.logical_and(pid >= start, pid < end)
            acc[...] = acc[...] + jnp.where(mask, o, 0.0)

    out_ref[...] = acc[...]


def _pair_add_body(p0_ref, p1_ref, w_ref, out_ref):
    out_ref[...] = (w_ref[:, 0:1] * p0_ref[0] + w_ref[:, 1:2] * p1_ref[0])


@jax.jit
def kernel(features, receptivity, gate_W1, gate_b1, gate_W2, gate_b2,
           exp_W1, exp_b1, exp_W2, exp_b2):
    gw1 = gate_W1.reshape(E, D_IN, D_HID)
    gb1 = gate_b1.reshape(1, D_HID)
    gb2 = gate_b2.reshape(1, E)
    rec = jnp.transpose(receptivity[..., 0], (1, 0))
    eb1 = exp_b1.reshape(E, 1, D_HID)
    eb2 = exp_b2.reshape(E, 1, D_OUT)

    tki, tkw, ranks_t, totals = pl.pallas_call(
        _gate_body,
        grid=(NT, E),
        in_specs=[
            pl.BlockSpec((1, TILE, D_IN), lambda i, e: (e, i, 0)),
            pl.BlockSpec((E, D_IN, D_HID), lambda i, e: (0, 0, 0)),
            pl.BlockSpec((D_HID, E), lambda i, e: (0, 0)),
            pl.BlockSpec((1, D_HID), lambda i, e: (0, 0)),
            pl.BlockSpec((1, E), lambda i, e: (0, 0)),
            pl.BlockSpec((TILE, E), lambda i, e: (i, 0)),
        ],
        out_specs=[
            pl.BlockSpec((TILE, TOP_K), lambda i, e: (i, 0)),
            pl.BlockSpec((TILE, TOP_K), lambda i, e: (i, 0)),
            pl.BlockSpec((TILE, E), lambda i, e: (i, 0)),
            pl.BlockSpec((1, E), lambda i, e: (0, 0)),
        ],
        out_shape=[
            jax.ShapeDtypeStruct((N, TOP_K), jnp.int32),
            jax.ShapeDtypeStruct((N, TOP_K), jnp.float32),
            jax.ShapeDtypeStruct((N, E), jnp.int32),
            jax.ShapeDtypeStruct((1, E), jnp.int32),
        ],
        scratch_shapes=[pltpu.VMEM((TILE, D_HID), jnp.float32)],
    )(features, gw1, gate_W2, gb1, gb2, rec)

    eidm = tki.reshape(SR, SL)
    posm, srcm, offs = pl.pallas_call(
        _route_body,
        grid=(1,),
        in_specs=[pl.BlockSpec((SR, SL), lambda i: (0, 0))],
        out_specs=[
            pl.BlockSpec((SR, SL), lambda i: (0, 0)),
            pl.BlockSpec((SR, SL), lambda i: (0, 0)),
            pl.BlockSpec((1, E), lambda i: (0, 0)),
        ],
        out_shape=[
            jax.ShapeDtypeStruct((SR, SL), jnp.int32),
            jax.ShapeDtypeStruct((SR, SL), jnp.int32),
            jax.ShapeDtypeStruct((1, E), jnp.int32),
        ],
    )(eidm)

    pos_i = posm
    gidx3 = srcm.reshape(NW, NCH, CHUNK)
    sidx3 = pos_i.reshape(NW, NCH, CHUNK)

    feat_flat = features.reshape(E * N, D_IN)
    compact_in = _sc_dispatch(feat_flat, gidx3, sidx3)

    offs9 = jnp.concatenate(
        [offs.reshape(E), jnp.full((1,), S, jnp.int32)])
    compact_out = pl.pallas_call(
        _grouped_body,
        grid_spec=pltpu.PrefetchScalarGridSpec(
            num_scalar_prefetch=1,
            grid=(NPT,),
            in_specs=[
                pl.BlockSpec((PT, D_IN), lambda t, offs: (t, 0)),
                pl.BlockSpec((E, D_IN, D_HID), lambda t, offs: (0, 0, 0)),
                pl.BlockSpec((E, 1, D_HID), lambda t, offs: (0, 0, 0)),
                pl.BlockSpec((E, D_HID, D_OUT), lambda t, offs: (0, 0, 0)),
                pl.BlockSpec((E, 1, D_OUT), lambda t, offs: (0, 0, 0)),
            ],
            out_specs=pl.BlockSpec((PT, D_OUT), lambda t, offs: (t, 0)),
            scratch_shapes=[pltpu.VMEM((PT, D_OUT), jnp.float32)],
        ),
        out_shape=jax.ShapeDtypeStruct((S, D_OUT), jnp.float32),
    )(offs9, compact_in, exp_W1, eb1, exp_W2, eb2)

    pos_kmaj3 = jnp.transpose(pos_i.reshape(N, TOP_K), (1, 0)).reshape(NW, NCH, CHUNK)
    pair = _sc_gather_rows(compact_out, pos_kmaj3, S, D_OUT)
    pair_k = pair.reshape(TOP_K, N, D_OUT)

    final_out = pl.pallas_call(
        _pair_add_body,
        grid=(NT,),
        in_specs=[
            pl.BlockSpec((1, TILE, D_OUT), lambda i: (0, i, 0)),
            pl.BlockSpec((1, TILE, D_OUT), lambda i: (1, i, 0)),
            pl.BlockSpec((TILE, TOP_K), lambda i: (i, 0)),
        ],
        out_specs=pl.BlockSpec((TILE, D_OUT), lambda i: (i, 0)),
        out_shape=jax.ShapeDtypeStruct((N, D_OUT), jnp.float32),
    )(pair_k, pair_k, tkw)

    return final_out, jnp.transpose(ranks_t, (1, 0)), totals.reshape(E)


# speedup vs baseline: 1.0016x; 1.0016x over previous
import functools

import jax
import jax.numpy as jnp
from jax import lax
from jax.experimental import pallas as pl
from jax.experimental.pallas import tpu as pltpu
from jax.experimental.pallas import tpu_sc as plsc

E = 8
TOP_K = 2
D_IN = 1024
D_HID = 512
D_OUT = 1024
N = 2048
TILE = 2048
NT = N // TILE
S = N * TOP_K
SR, SL = 32, 128
PT = 512
NPT = S // PT

NW = 32
ROWS_PER_W = S // NW
CHUNK = 32
NCH = ROWS_PER_W // CHUNK

_NEG_INF = float("-inf")


def _gate_body(feat_ref, gw1_ref, gw2_ref, gb1_ref, gb2_ref, rec_ref,
               tki_ref, tkw_ref, ranks_ref, totals_ref, gh_acc):
    i = pl.program_id(0)
    e = pl.program_id(1)
    part = jnp.dot(feat_ref[0], gw1_ref[e], preferred_element_type=jnp.float32)

    @pl.when(e == 0)
    def _():
        gh_acc[...] = part

    @pl.when(e > 0)
    def _():
        gh_acc[...] = gh_acc[...] + part

    @pl.when(e == E - 1)
    def _():
        gh = jax.nn.relu(gh_acc[...] + gb1_ref[...])
        logits = jnp.dot(gh, gw2_ref[...],
                         preferred_element_type=jnp.float32) + gb2_ref[...]
        m = jnp.max(logits, axis=1, keepdims=True)
        ex = jnp.exp(logits - m)
        sm = ex / jnp.sum(ex, axis=1, keepdims=True)
        scores = sm + rec_ref[...]

        iota = lax.broadcasted_iota(jnp.int32, (TILE, E), 1)
        v1 = jnp.max(scores, axis=1, keepdims=True)
        i1 = jnp.min(jnp.where(scores == v1, iota, E), axis=1, keepdims=True)
        masked = jnp.where(iota == i1, _NEG_INF, scores)
        v2 = jnp.max(masked, axis=1, keepdims=True)
        i2 = jnp.min(jnp.where(masked == v2, iota, E), axis=1, keepdims=True)
        s = v1 + v2
        w_1 = v1 / s
        w_2 = v2 / s

        tki_ref[...] = jnp.concatenate([i1, i2], axis=1)
        tkw_ref[...] = jnp.concatenate([w_1, w_2], axis=1)
        oh1 = (iota == i1)
        oh2 = (iota == i2)
        r = 2 - 2 * oh1.astype(jnp.int32) - oh2.astype(jnp.int32)
        ranks_ref[...] = r.T
        part_tot = jnp.sum(r, axis=0, keepdims=True)

        @pl.when(i == 0)
        def _():
            totals_ref[...] = part_tot

        @pl.when(i > 0)
        def _():
            totals_ref[...] = totals_ref[...] + part_tot


def _route_body(eid_ref, posm_ref, srcm_ref, offs_ref):
    eid = eid_ref[...]
    jl = lax.broadcasted_iota(jnp.int32, (SL, SL), 1)
    kl = lax.broadcasted_iota(jnp.int32, (SL, SL), 0)
    t128 = (kl <= jl).astype(jnp.float32)
    rr = lax.broadcasted_iota(jnp.int32, (SR, SR), 0)
    rc = lax.broadcasted_iota(jnp.int32, (SR, SR), 1)
    t32s = (rc < rr).astype(jnp.float32)

    p1 = []
    scols = []
    for e in range(E):
        oh = (eid == e).astype(jnp.float32)
        pe = jnp.dot(oh, t128, preferred_element_type=jnp.float32)
        p1.append(pe)
        scols.append(pe[:, SL - 1:SL])
    smat = jnp.concatenate(scols, axis=1)
    omat = jnp.dot(t32s, smat, preferred_element_type=jnp.float32)
    counts = jnp.sum(smat, axis=0, keepdims=True)

    offs_list = [jnp.zeros((1, 1), jnp.float32)]
    for e in range(1, E):
        offs_list.append(offs_list[e - 1] + counts[:, e - 1:e])
    offs = jnp.concatenate(offs_list, axis=1)

    pos = jnp.zeros((SR, SL), jnp.float32)
    for e in range(E):
        oh = (eid == e).astype(jnp.float32)
        pos = pos + oh * (p1[e] - 1.0 + omat[:, e:e + 1] + offs[:, e:e + 1])
    posm_ref[...] = pos.astype(jnp.int32)

    ri = lax.broadcasted_iota(jnp.int32, (SR, SL), 0)
    ji = lax.broadcasted_iota(jnp.int32, (SR, SL), 1)
    tok = (ri * SL + ji) // TOP_K
    srcm_ref[...] = eid * N + tok
    offs_ref[...] = offs.astype(jnp.int32)


def _sc_dispatch(table, gidx3, sidx3):
    mesh = plsc.VectorSubcoreMesh(core_axis_name="c", subcore_axis_name="s")

    @functools.partial(
        pl.kernel, mesh=mesh,
        out_type=jax.ShapeDtypeStruct((S, D_IN), jnp.float32),
        scratch_types=[
            pltpu.VMEM((NCH, CHUNK), jnp.int32),
            pltpu.VMEM((NCH, CHUNK), jnp.int32),
            pltpu.VMEM((CHUNK, D_IN), jnp.float32),
            pltpu.VMEM((CHUNK, D_IN), jnp.float32),
            pltpu.VMEM((CHUNK, D_IN), jnp.float32),
            pltpu.SemaphoreType.DMA,
            pltpu.SemaphoreType.DMA,
            pltpu.SemaphoreType.DMA,
            pltpu.SemaphoreType.DMA,
            pltpu.SemaphoreType.DMA,
            pltpu.SemaphoreType.DMA,
        ],
    )
    def k(table_hbm, gidx_hbm, sidx_hbm, out_hbm,
          gidx_v, sidx_v, b0, b1, b2, g0, g1, g2, s0, s1, s2):
        wid = lax.axis_index("s") * 2 + lax.axis_index("c")
        pltpu.sync_copy(gidx_hbm.at[wid], gidx_v)
        pltpu.sync_copy(sidx_hbm.at[wid], sidx_v)
        bufs = (b0, b1, b2)
        gsems = (g0, g1, g2)
        ssems = (s0, s1, s2)
        gcps = [None] * 3
        scps = [None] * 3
        for c in range(NCH):
            b = c % 3
            if c >= 3:
                scps[b].wait()
            gcps[b] = pltpu.async_copy(table_hbm.at[gidx_v.at[c]], bufs[b],
                                       gsems[b])
            if c >= 1:
                pb = (c - 1) % 3
                gcps[pb].wait()
                scps[pb] = pltpu.async_copy(bufs[pb],
                                            out_hbm.at[sidx_v.at[c - 1]],
                                            ssems[pb])
        lb = (NCH - 1) % 3
        gcps[lb].wait()
        scps[lb] = pltpu.async_copy(bufs[lb], out_hbm.at[sidx_v.at[NCH - 1]],
                                    ssems[lb])
        for c in range(max(0, NCH - 3), NCH):
            scps[c % 3].wait()

    return k(table, gidx3, sidx3)


def _sc_gather_rows(table, idx3, out_rows, out_cols):
    mesh = plsc.VectorSubcoreMesh(core_axis_name="c", subcore_axis_name="s")

    @functools.partial(
        pl.kernel, mesh=mesh,
        out_type=jax.ShapeDtypeStruct((out_rows, out_cols), jnp.float32),
        scratch_types=[
            pltpu.VMEM((NCH, CHUNK), jnp.int32),
            pltpu.VMEM((CHUNK, out_cols), jnp.float32),
            pltpu.VMEM((CHUNK, out_cols), jnp.float32),
            pltpu.VMEM((CHUNK, out_cols), jnp.float32),
            pltpu.SemaphoreType.DMA,
            pltpu.SemaphoreType.DMA,
            pltpu.SemaphoreType.DMA,
            pltpu.SemaphoreType.DMA,
            pltpu.SemaphoreType.DMA,
            pltpu.SemaphoreType.DMA,
        ],
    )
    def k(table_hbm, idx_hbm, out_hbm, idx_v, b0, b1, b2,
          g0, g1, g2, w0, w1, w2):
        wid = lax.axis_index("s") * 2 + lax.axis_index("c")
        base = wid * ROWS_PER_W
        pltpu.sync_copy(idx_hbm.at[wid], idx_v)
        bufs = (b0, b1, b2)
        gsems = (g0, g1, g2)
        wsems = (w0, w1, w2)
        gcps = [None] * 3
        wcps = [None] * 3
        for c in range(NCH):
            b = c % 3
            if c >= 3:
                wcps[b].wait()
            gcps[b] = pltpu.async_copy(table_hbm.at[idx_v.at[c]], bufs[b],
                                       gsems[b])
            if c >= 1:
                pb = (c - 1) % 3
                gcps[pb].wait()
                wcps[pb] = pltpu.async_copy(
                    bufs[pb], out_hbm.at[pl.ds(base + (c - 1) * CHUNK, CHUNK)],
                    wsems[pb])
        lb = (NCH - 1) % 3
        gcps[lb].wait()
        wcps[lb] = pltpu.async_copy(
            bufs[lb], out_hbm.at[pl.ds(base + (NCH - 1) * CHUNK, CHUNK)],
            wsems[lb])
        for c in range(max(0, NCH - 3), NCH):
            wcps[c % 3].wait()

    return k(table, idx3)


def _grouped_body(offs_ref, cin_ref, w1_ref, b1_ref, w2_ref, b2_ref, out_ref,
                  acc):
    t = pl.program_id(0)
    pid = t * PT + lax.broadcasted_iota(jnp.int32, (PT, 1), 0)
    x = cin_ref[...]
    acc[...] = jnp.zeros((PT, D_OUT), jnp.float32)
    for e in range(E):
        start = offs_ref[e]
        end = offs_ref[e + 1]
        active = jnp.logical_and(start < (t + 1) * PT, end > t * PT)

        @pl.when(active)
        def _(e=e, start=start, end=end):
            h = jax.nn.relu(
                jnp.dot(x, w1_ref[e],
                        preferred_element_type=jnp.float32) + b1_ref[e])
            o = jnp.dot(h, w2_ref[e],
                        preferred_element_type=jnp.float32) + b2_ref[e]
            mask = jnp.logical_and(pid >= start, pid < end)
            acc[...] = acc[...] + jnp.where(mask, o, 0.0)

    out_ref[...] = acc[...]


def _pair_add_body(p0_ref, p1_ref, w_ref, out_ref):
    out_ref[...] = (w_ref[:, 0:1] * p0_ref[0] + w_ref[:, 1:2] * p1_ref[0])


@jax.jit
def kernel(features, receptivity, gate_W1, gate_b1, gate_W2, gate_b2,
           exp_W1, exp_b1, exp_W2, exp_b2):
    gw1 = gate_W1.reshape(E, D_IN, D_HID)
    gb1 = gate_b1.reshape(1, D_HID)
    gb2 = gate_b2.reshape(1, E)
    rec = jnp.transpose(receptivity[..., 0], (1, 0))
    eb1 = exp_b1.reshape(E, 1, D_HID)
    eb2 = exp_b2.reshape(E, 1, D_OUT)

    tki, tkw, ranks_t, totals = pl.pallas_call(
        _gate_body,
        grid=(NT, E),
        in_specs=[
            pl.BlockSpec((1, TILE, D_IN), lambda i, e: (e, i, 0)),
            pl.BlockSpec((E, D_IN, D_HID), lambda i, e: (0, 0, 0)),
            pl.BlockSpec((D_HID, E), lambda i, e: (0, 0)),
            pl.BlockSpec((1, D_HID), lambda i, e: (0, 0)),
            pl.BlockSpec((1, E), lambda i, e: (0, 0)),
            pl.BlockSpec((TILE, E), lambda i, e: (i, 0)),
        ],
        out_specs=[
            pl.BlockSpec((TILE, TOP_K), lambda i, e: (i, 0)),
            pl.BlockSpec((TILE, TOP_K), lambda i, e: (i, 0)),
            pl.BlockSpec((E, TILE), lambda i, e: (0, i)),
            pl.BlockSpec((1, E), lambda i, e: (0, 0)),
        ],
        out_shape=[
            jax.ShapeDtypeStruct((N, TOP_K), jnp.int32),
            jax.ShapeDtypeStruct((N, TOP_K), jnp.float32),
            jax.ShapeDtypeStruct((E, N), jnp.int32),
            jax.ShapeDtypeStruct((1, E), jnp.int32),
        ],
        scratch_shapes=[pltpu.VMEM((TILE, D_HID), jnp.float32)],
    )(features, gw1, gate_W2, gb1, gb2, rec)

    eidm = tki.reshape(SR, SL)
    posm, srcm, offs = pl.pallas_call(
        _route_body,
        grid=(1,),
        in_specs=[pl.BlockSpec((SR, SL), lambda i: (0, 0))],
        out_specs=[
            pl.BlockSpec((SR, SL), lambda i: (0, 0)),
            pl.BlockSpec((SR, SL), lambda i: (0, 0)),
            pl.BlockSpec((1, E), lambda i: (0, 0)),
        ],
        out_shape=[
            jax.ShapeDtypeStruct((SR, SL), jnp.int32),
            jax.ShapeDtypeStruct((SR, SL), jnp.int32),
            jax.ShapeDtypeStruct((1, E), jnp.int32),
        ],
    )(eidm)

    pos_i = posm
    gidx3 = srcm.reshape(NW, NCH, CHUNK)
    sidx3 = pos_i.reshape(NW, NCH, CHUNK)

    feat_flat = features.reshape(E * N, D_IN)
    compact_in = _sc_dispatch(feat_flat, gidx3, sidx3)

    offs9 = jnp.concatenate(
        [offs.reshape(E), jnp.full((1,), S, jnp.int32)])
    compact_out = pl.pallas_call(
        _grouped_body,
        grid_spec=pltpu.PrefetchScalarGridSpec(
            num_scalar_prefetch=1,
            grid=(NPT,),
            in_specs=[
                pl.BlockSpec((PT, D_IN), lambda t, offs: (t, 0)),
                pl.BlockSpec((E, D_IN, D_HID), lambda t, offs: (0, 0, 0)),
                pl.BlockSpec((E, 1, D_HID), lambda t, offs: (0, 0, 0)),
                pl.BlockSpec((E, D_HID, D_OUT), lambda t, offs: (0, 0, 0)),
                pl.BlockSpec((E, 1, D_OUT), lambda t, offs: (0, 0, 0)),
            ],
            out_specs=pl.BlockSpec((PT, D_OUT), lambda t, offs: (t, 0)),
            scratch_shapes=[pltpu.VMEM((PT, D_OUT), jnp.float32)],
        ),
        out_shape=jax.ShapeDtypeStruct((S, D_OUT), jnp.float32),
    )(offs9, compact_in, exp_W1, eb1, exp_W2, eb2)

    pos_kmaj3 = jnp.transpose(pos_i.reshape(N, TOP_K), (1, 0)).reshape(NW, NCH, CHUNK)
    pair = _sc_gather_rows(compact_out, pos_kmaj3, S, D_OUT)
    pair_k = pair.reshape(TOP_K, N, D_OUT)

    final_out = pl.pallas_call(
        _pair_add_body,
        grid=(NT,),
        in_specs=[
            pl.BlockSpec((1, TILE, D_OUT), lambda i: (0, i, 0)),
            pl.BlockSpec((1, TILE, D_OUT), lambda i: (1, i, 0)),
            pl.BlockSpec((TILE, TOP_K), lambda i: (i, 0)),
        ],
        out_specs=pl.BlockSpec((TILE, D_OUT), lambda i: (i, 0)),
        out_shape=jax.ShapeDtypeStruct((N, D_OUT), jnp.float32),
    )(pair_k, pair_k, tkw)

    return final_out, ranks_t, totals.reshape(E)
